# baseline (speedup 1.0000x reference)
import functools

import jax
import jax.numpy as jnp
import numpy as np
from jax import lax
from jax.experimental import pallas as pl
from jax.experimental.pallas import tpu as pltpu
from jax.experimental.pallas import tpu_sc as plsc

NC = 2
NS = 16
NW = NC * NS
K = 80
L = 16

def _mm_bf16_kernel(x_ref, w_ref, o_ref):
    o_ref[...] = jnp.dot(x_ref[...], w_ref[...],
                         preferred_element_type=jnp.float32).astype(jnp.bfloat16)


def _rela_proj_kernel(x_ref, wr_ref, wq_ref, b_ref, or_ref, oq_ref):
    x = x_ref[...]
    or_ref[...] = jnp.dot(x, wr_ref[...],
                          preferred_element_type=jnp.float32).astype(jnp.bfloat16)
    oq_ref[...] = (jnp.dot(x, wq_ref[...], preferred_element_type=jnp.float32)
                   + b_ref[...]).astype(jnp.bfloat16)


def _final_kernel(p_ref, w_ref, o_ref):
    o_ref[...] = jnp.dot(p_ref[0] + p_ref[1], w_ref[...],
                         preferred_element_type=jnp.float32)


def _sc_cq_kernel(qrel_h, qp_h, cq_h, qrel_v, cq_v, sem):
    c = lax.axis_index("c")
    s = lax.axis_index("s")

    @pl.when(jnp.logical_and(c == 0, s == 0))
    def _():
        pltpu.sync_copy(qrel_h, qrel_v)
        pltpu.async_copy(qp_h.at[qrel_v], cq_v, sem).wait()
        pltpu.sync_copy(cq_v, cq_h)


def _halves(xi):
    a = plsc.bitcast(lax.shift_left(xi, 16), jnp.float32)
    b = plsc.bitcast(lax.bitwise_and(xi, jnp.int32(-65536)), jnp.float32)
    return a, b


def _sc_edge_kernel(n_node, n_chunk, idx4_h, hs_h, rl_h, cq_h, wp_h, zero_h,
                    out_h, idx_b, wp_v, hs_b, rl_b, qp_b, msg_b,
                    acc, sem_g, sem_i, sem_s):
    c = lax.axis_index("c")
    s = lax.axis_index("s")
    wid = s * NC + c

    rows = (n_node // NS) & ~7
    tail = n_node - NS * rows
    pltpu.sync_copy(zero_h.at[pl.ds(s * rows, rows)], acc.at[pl.ds(s * rows, rows)])
    if tail:
        @pl.when(s == 0)
        def _zero_tail():
            pltpu.sync_copy(zero_h.at[pl.ds(NS * rows, tail)],
                            acc.at[pl.ds(NS * rows, tail)])
    pltpu.sync_copy(wp_h, wp_v)
    plsc.subcore_barrier()

    wa0 = wp_v[pl.ds(0, L)]
    wb0 = wp_v[pl.ds(L, L)]
    wa1 = wp_v[pl.ds(2 * L, L)]
    wb1 = wp_v[pl.ds(3 * L, L)]
    bias = wp_v[pl.ds(4 * L, L)]

    n_mine = (n_chunk - 1 - wid) // NW + 1

    def issue_idx(slot, j):
        base = (wid + j * NW) * (4 * K)
        return pltpu.async_copy(idx4_h.at[pl.ds(base, 4 * K)],
                                idx_b.at[slot], sem_i)

    def compute(slot):
        @plsc.parallel_loop(0, K, unroll=2)
        def edge_body(e):
            zero = jnp.float32(0)
            xh0 = hs_b[slot, e, pl.ds(64, L)]
            xr0 = rl_b[slot, e, pl.ds(64, L)]
            xq0 = qp_b[slot, e, pl.ds(0, L)]
            ah0, bh0 = _halves(xh0)
            ar0, br0 = _halves(xr0)
            aq0, bq0 = _halves(xq0)
            pa0 = ah0 + ar0 + aq0
            pb0 = bh0 + br0 + bq0
            xh1 = hs_b[slot, e, pl.ds(64 + L, L)]
            xr1 = rl_b[slot, e, pl.ds(64 + L, L)]
            xq1 = qp_b[slot, e, pl.ds(L, L)]
            ah1, bh1 = _halves(xh1)
            ar1, br1 = _halves(xr1)
            aq1, bq1 = _halves(xq1)
            pa1 = ah1 + ar1 + aq1
            pb1 = bh1 + br1 + bq1
            t0 = jnp.maximum(pa0, zero) * wa0
            t1 = jnp.maximum(pb0, zero) * wb0
            t2 = jnp.maximum(pa1, zero) * wa1
            t3 = jnp.maximum(pb1, zero) * wb1
            tot = jnp.sum((t0 + t1) + (t2 + t3))
            x = lax.broadcast_in_dim(tot, (L,), ()) + bias
            alpha = 1.0 / (1.0 + jnp.exp(-x))
            for t in range(4):
                xh = hs_b[slot, e, pl.ds(t * L, L)]
                xr = rl_b[slot, e, pl.ds(t * L, L)]
                ah, bh = _halves(xh)
                ar, br = _halves(xr)
                msg_b[e, pl.ds(2 * t * L, L)] = ah * ar * alpha
                msg_b[e, pl.ds((2 * t + 1) * L, L)] = bh * br * alpha

    def scatter(slot):
        return pltpu.async_copy(msg_b, acc.at[idx_b.at[slot, pl.ds(3 * K, K)]],
                                sem_s, add=True)

    issue_idx(0, 0).wait()

    def chunk_body(j, carry):
        p = j & 1
        q = 1 - p
        g1 = pltpu.async_copy(hs_h.at[idx_b.at[p, pl.ds(0, K)]],
                              hs_b.at[p], sem_g)
        g2 = pltpu.async_copy(rl_h.at[idx_b.at[p, pl.ds(K, K)]],
                              rl_b.at[p], sem_g)
        g3 = pltpu.async_copy(cq_h.at[idx_b.at[p, pl.ds(2 * K, K)]],
                              qp_b.at[p], sem_g)

        @pl.when(j > 0)
        def _compute_prev():
            compute(q)
            sc_h = scatter(q)
            g1.wait()
            g2.wait()
            g3.wait()
            sc_h.wait()

        @pl.when(j == 0)
        def _first_waits():
            g1.wait()
            g2.wait()
            g3.wait()

        @pl.when(j + 1 < n_mine)
        def _prefetch_idx():
            issue_idx(q, j + 1).wait()

        return carry

    lax.fori_loop(0, n_mine, chunk_body, 0)
    last = (n_mine - 1) & 1
    compute(last)
    scatter(last).wait()

    plsc.subcore_barrier()
    pltpu.sync_copy(acc.at[pl.ds(s * rows, rows)],
                    out_h.at[pl.ds(c * n_node + s * rows, rows)])
    if tail:
        @pl.when(s == 0)
        def _out_tail():
            pltpu.sync_copy(acc.at[pl.ds(NS * rows, tail)],
                            out_h.at[pl.ds(c * n_node + NS * rows, tail)])


def _pack_i32(x_bf16):
    r, cc = x_bf16.shape
    return lax.bitcast_convert_type(x_bf16.reshape(r, cc // 2, 2), jnp.int32)


def _evens_odds_perm(width):
    perm = []
    for t in range(width // 32):
        perm += [32 * t + 2 * k for k in range(16)]
        perm += [32 * t + 2 * k + 1 for k in range(16)]
    return perm


def kernel(q_sub, q_rel, r_idx, hidden, edges, n_node, rela_embed, Ws_attn,
           Wr_attn, Wqr_W, Wqr_b, walpha_W, walpha_b, W_h):
    del q_sub
    n, d = hidden.shape
    v = rela_embed.shape[0]
    e = r_idx.shape[0]
    assert e % K == 0
    n_chunk = e // K

    e32 = edges.astype(jnp.int32)
    sub_i = e32[:, 0]
    rel_i = e32[:, 1]
    obj_i = jnp.minimum(e32[:, 2], jnp.int32(n_node) - 1)
    ridx_i = r_idx.astype(jnp.int32)
    qrel_i = q_rel.astype(jnp.int32)
    idx4 = jnp.stack([sub_i.reshape(n_chunk, K), rel_i.reshape(n_chunk, K),
                      ridx_i.reshape(n_chunk, K), obj_i.reshape(n_chunk, K)],
                     axis=1).reshape(-1)

    p64 = np.array(_evens_odds_perm(64), np.int32)
    wp = jnp.concatenate([walpha_W[p64, 0],
                          jnp.broadcast_to(walpha_b, (L,))]).astype(jnp.float32)

    hs_proj = pl.pallas_call(
        _mm_bf16_kernel,
        grid=(10,),
        in_specs=[pl.BlockSpec((n // 10, d), lambda i: (i, 0)),
                  pl.BlockSpec((d, 64), lambda i: (0, 0))],
        out_specs=pl.BlockSpec((n // 10, 64), lambda i: (i, 0)),
        out_shape=jax.ShapeDtypeStruct((n, 64), jnp.bfloat16),
    )(hidden, Ws_attn)

    rb = 1024
    rl_proj, qp_proj = pl.pallas_call(
        _rela_proj_kernel,
        grid=(pl.cdiv(v, rb),),
        in_specs=[pl.BlockSpec((rb, d), lambda i: (i, 0)),
                  pl.BlockSpec((d, 64), lambda i: (0, 0)),
                  pl.BlockSpec((d, 64), lambda i: (0, 0)),
                  pl.BlockSpec((1, 64), lambda i: (0, 0))],
        out_specs=[pl.BlockSpec((rb, 64), lambda i: (i, 0)),
                   pl.BlockSpec((rb, 64), lambda i: (i, 0))],
        out_shape=[jax.ShapeDtypeStruct((v, 64), jnp.bfloat16),
                   jax.ShapeDtypeStruct((v, 64), jnp.bfloat16)],
    )(rela_embed, Wr_attn, Wqr_W, Wqr_b.reshape(1, 64))

    hs_tab = jnp.concatenate([_pack_i32(hidden.astype(jnp.bfloat16)),
                              _pack_i32(hs_proj)], axis=1)
    rl_tab = jnp.concatenate([_pack_i32(rela_embed.astype(jnp.bfloat16)),
                              _pack_i32(rl_proj)], axis=1)
    qp_i = _pack_i32(qp_proj)

    cq = pl.kernel(
        _sc_cq_kernel,
        out_type=jax.ShapeDtypeStruct((64, 32), jnp.int32),
        mesh=plsc.VectorSubcoreMesh(core_axis_name="c", subcore_axis_name="s"),
        scratch_types=[
            pltpu.VMEM((64,), jnp.int32),
            pltpu.VMEM((64, 32), jnp.int32),
            pltpu.SemaphoreType.DMA,
        ],
        compiler_params=pltpu.CompilerParams(use_tc_tiling_on_sc=False,
                                             needs_layout_passes=False),
    )(qrel_i, qp_i)

    sc = pl.kernel(
        functools.partial(_sc_edge_kernel, n, n_chunk),
        out_type=jax.ShapeDtypeStruct((NC * n, d), jnp.float32),
        mesh=plsc.VectorSubcoreMesh(core_axis_name="c", subcore_axis_name="s"),
        scratch_types=[
            pltpu.VMEM((2, 4 * K), jnp.int32),
            pltpu.VMEM((5 * L,), jnp.float32),
            pltpu.VMEM((2, K, 96), jnp.int32),
            pltpu.VMEM((2, K, 96), jnp.int32),
            pltpu.VMEM((2, K, 32), jnp.int32),
            pltpu.VMEM((K, d), jnp.float32),
            pltpu.VMEM_SHARED((n, d), jnp.float32),
            pltpu.SemaphoreType.DMA,
            pltpu.SemaphoreType.DMA,
            pltpu.SemaphoreType.DMA,
        ],
        compiler_params=pltpu.CompilerParams(use_tc_tiling_on_sc=False,
                                             needs_layout_passes=False),
    )
    partial_out = sc(idx4, hs_tab, rl_tab, cq, wp, jnp.zeros((n, d), jnp.float32))

    p128 = np.array(_evens_odds_perm(d), np.int32)
    w_h_perm = W_h[p128, :]
    p = partial_out.reshape(NC, n, d)
    fb = 1000
    hidden_new = pl.pallas_call(
        _final_kernel,
        grid=(n // fb,),
        in_specs=[pl.BlockSpec((NC, fb, d), lambda i: (0, i, 0)),
                  pl.BlockSpec((d, d), lambda i: (0, 0))],
        out_specs=pl.BlockSpec((fb, d), lambda i: (i, 0)),
        out_shape=jax.ShapeDtypeStruct((n, d), jnp.float32),
    )(p, w_h_perm)
    return hidden_new


# speedup vs baseline: 1.0120x; 1.0120x over previous
import functools

import jax
import jax.numpy as jnp
import numpy as np
from jax import lax
from jax.experimental import pallas as pl
from jax.experimental.pallas import tpu as pltpu
from jax.experimental.pallas import tpu_sc as plsc

NC = 2
NS = 16
NW = NC * NS
K = 80
L = 16

def _mm_bf16_kernel(x_ref, w_ref, o_ref):
    o_ref[...] = jnp.dot(x_ref[...], w_ref[...],
                         preferred_element_type=jnp.float32).astype(jnp.bfloat16)


def _rela_proj_kernel(x_ref, wr_ref, wq_ref, b_ref, or_ref, oq_ref):
    x = x_ref[...]
    or_ref[...] = jnp.dot(x, wr_ref[...],
                          preferred_element_type=jnp.float32).astype(jnp.bfloat16)
    oq_ref[...] = (jnp.dot(x, wq_ref[...], preferred_element_type=jnp.float32)
                   + b_ref[...]).astype(jnp.bfloat16)


def _final_kernel(p_ref, w_ref, o_ref):
    o_ref[...] = jnp.dot(p_ref[0] + p_ref[1], w_ref[...],
                         preferred_element_type=jnp.float32)


def _sc_cq_kernel(qrel_h, qp_h, cq_h, qrel_v, cq_v, sem):
    c = lax.axis_index("c")
    s = lax.axis_index("s")

    @pl.when(jnp.logical_and(c == 0, s == 0))
    def _():
        pltpu.sync_copy(qrel_h, qrel_v)
        pltpu.async_copy(qp_h.at[qrel_v], cq_v, sem).wait()
        pltpu.sync_copy(cq_v, cq_h)


def _halves(xi):
    a = plsc.bitcast(lax.shift_left(xi, 16), jnp.float32)
    b = plsc.bitcast(lax.bitwise_and(xi, jnp.int32(-65536)), jnp.float32)
    return a, b


def _sc_edge_kernel(n_node, n_chunk, idx4_h, hs_h, rl_h, cq_h, wp_h, zero_h,
                    out_h, idx_b, obj_s, wp_v, hs_b, rl_b, qp_b, msg_b,
                    acc, sem_g, sem_i, sem_s):
    c = lax.axis_index("c")
    s = lax.axis_index("s")
    wid = s * NC + c

    rows = (n_node // NS) & ~7
    tail = n_node - NS * rows
    pltpu.sync_copy(zero_h.at[pl.ds(s * rows, rows)], acc.at[pl.ds(s * rows, rows)])
    if tail:
        @pl.when(s == 0)
        def _zero_tail():
            pltpu.sync_copy(zero_h.at[pl.ds(NS * rows, tail)],
                            acc.at[pl.ds(NS * rows, tail)])
    pltpu.sync_copy(wp_h, wp_v)
    plsc.subcore_barrier()

    wa0 = wp_v[pl.ds(0, L)]
    wb0 = wp_v[pl.ds(L, L)]
    wa1 = wp_v[pl.ds(2 * L, L)]
    wb1 = wp_v[pl.ds(3 * L, L)]
    bias = wp_v[pl.ds(4 * L, L)]

    n_mine = (n_chunk - 1 - wid) // NW + 1

    def issue_idx(slot, j):
        base = (wid + j * NW) * (4 * K)
        return pltpu.async_copy(idx4_h.at[pl.ds(base, 4 * K)],
                                idx_b.at[slot], sem_i)

    def compute(slot):
        @plsc.parallel_loop(0, K, unroll=2)
        def edge_body(e):
            zero = jnp.float32(0)
            xh0 = hs_b[slot, e, pl.ds(64, L)]
            xr0 = rl_b[slot, e, pl.ds(64, L)]
            xq0 = qp_b[slot, e, pl.ds(0, L)]
            ah0, bh0 = _halves(xh0)
            ar0, br0 = _halves(xr0)
            aq0, bq0 = _halves(xq0)
            pa0 = ah0 + ar0 + aq0
            pb0 = bh0 + br0 + bq0
            xh1 = hs_b[slot, e, pl.ds(64 + L, L)]
            xr1 = rl_b[slot, e, pl.ds(64 + L, L)]
            xq1 = qp_b[slot, e, pl.ds(L, L)]
            ah1, bh1 = _halves(xh1)
            ar1, br1 = _halves(xr1)
            aq1, bq1 = _halves(xq1)
            pa1 = ah1 + ar1 + aq1
            pb1 = bh1 + br1 + bq1
            t0 = jnp.maximum(pa0, zero) * wa0
            t1 = jnp.maximum(pb0, zero) * wb0
            t2 = jnp.maximum(pa1, zero) * wa1
            t3 = jnp.maximum(pb1, zero) * wb1
            tot = jnp.sum((t0 + t1) + (t2 + t3))
            x = lax.broadcast_in_dim(tot, (L,), ()) + bias
            alpha = 1.0 / (1.0 + jnp.exp(-x))
            for t in range(4):
                xh = hs_b[slot, e, pl.ds(t * L, L)]
                xr = rl_b[slot, e, pl.ds(t * L, L)]
                ah, bh = _halves(xh)
                ar, br = _halves(xr)
                msg_b[e, pl.ds(2 * t * L, L)] = ah * ar * alpha
                msg_b[e, pl.ds((2 * t + 1) * L, L)] = bh * br * alpha
        for t in range(K // L):
            obj_s[slot, pl.ds(t * L, L)] = idx_b[slot, pl.ds(3 * K + t * L, L)]

    def scatter(slot):
        return pltpu.async_copy(msg_b, acc.at[obj_s.at[slot]], sem_s, add=True)

    issue_idx(0, 0).wait()

    def chunk_body(j, carry):
        p = j & 1
        q = 1 - p
        g1 = pltpu.async_copy(hs_h.at[idx_b.at[p, pl.ds(0, K)]],
                              hs_b.at[p], sem_g)
        g2 = pltpu.async_copy(rl_h.at[idx_b.at[p, pl.ds(K, K)]],
                              rl_b.at[p], sem_g)
        g3 = pltpu.async_copy(cq_h.at[idx_b.at[p, pl.ds(2 * K, K)]],
                              qp_b.at[p], sem_g)

        @pl.when(j > 0)
        def _steady():
            compute(q)
            sc_h = scatter(q)

            @pl.when(j + 1 < n_mine)
            def _prefetch_idx():
                issue_idx(q, j + 1).wait()

            g1.wait()
            g2.wait()
            g3.wait()
            sc_h.wait()

        @pl.when(j == 0)
        def _first():
            @pl.when(n_mine > 1)
            def _prefetch_idx0():
                issue_idx(q, 1).wait()

            g1.wait()
            g2.wait()
            g3.wait()

        return carry

    lax.fori_loop(0, n_mine, chunk_body, 0)
    last = (n_mine - 1) & 1
    compute(last)
    scatter(last).wait()

    plsc.subcore_barrier()
    pltpu.sync_copy(acc.at[pl.ds(s * rows, rows)],
                    out_h.at[pl.ds(c * n_node + s * rows, rows)])
    if tail:
        @pl.when(s == 0)
        def _out_tail():
            pltpu.sync_copy(acc.at[pl.ds(NS * rows, tail)],
                            out_h.at[pl.ds(c * n_node + NS * rows, tail)])


def _pack_i32(x_bf16):
    r, cc = x_bf16.shape
    return lax.bitcast_convert_type(x_bf16.reshape(r, cc // 2, 2), jnp.int32)


def _evens_odds_perm(width):
    perm = []
    for t in range(width // 32):
        perm += [32 * t + 2 * k for k in range(16)]
        perm += [32 * t + 2 * k + 1 for k in range(16)]
    return perm


def kernel(q_sub, q_rel, r_idx, hidden, edges, n_node, rela_embed, Ws_attn,
           Wr_attn, Wqr_W, Wqr_b, walpha_W, walpha_b, W_h):
    del q_sub
    n, d = hidden.shape
    v = rela_embed.shape[0]
    e = r_idx.shape[0]
    assert e % K == 0
    n_chunk = e // K

    e32 = edges.astype(jnp.int32)
    sub_i = e32[:, 0]
    rel_i = e32[:, 1]
    obj_i = jnp.minimum(e32[:, 2], jnp.int32(n_node) - 1)
    ridx_i = r_idx.astype(jnp.int32)
    qrel_i = q_rel.astype(jnp.int32)
    idx4 = jnp.stack([sub_i.reshape(n_chunk, K), rel_i.reshape(n_chunk, K),
                      ridx_i.reshape(n_chunk, K), obj_i.reshape(n_chunk, K)],
                     axis=1).reshape(-1)

    p64 = np.array(_evens_odds_perm(64), np.int32)
    wp = jnp.concatenate([walpha_W[p64, 0],
                          jnp.broadcast_to(walpha_b, (L,))]).astype(jnp.float32)

    hs_proj = pl.pallas_call(
        _mm_bf16_kernel,
        grid=(10,),
        in_specs=[pl.BlockSpec((n // 10, d), lambda i: (i, 0)),
                  pl.BlockSpec((d, 64), lambda i: (0, 0))],
        out_specs=pl.BlockSpec((n // 10, 64), lambda i: (i, 0)),
        out_shape=jax.ShapeDtypeStruct((n, 64), jnp.bfloat16),
    )(hidden, Ws_attn)

    rb = 1024
    rl_proj, qp_proj = pl.pallas_call(
        _rela_proj_kernel,
        grid=(pl.cdiv(v, rb),),
        in_specs=[pl.BlockSpec((rb, d), lambda i: (i, 0)),
                  pl.BlockSpec((d, 64), lambda i: (0, 0)),
                  pl.BlockSpec((d, 64), lambda i: (0, 0)),
                  pl.BlockSpec((1, 64), lambda i: (0, 0))],
        out_specs=[pl.BlockSpec((rb, 64), lambda i: (i, 0)),
                   pl.BlockSpec((rb, 64), lambda i: (i, 0))],
        out_shape=[jax.ShapeDtypeStruct((v, 64), jnp.bfloat16),
                   jax.ShapeDtypeStruct((v, 64), jnp.bfloat16)],
    )(rela_embed, Wr_attn, Wqr_W, Wqr_b.reshape(1, 64))

    hs_tab = jnp.concatenate([_pack_i32(hidden.astype(jnp.bfloat16)),
                              _pack_i32(hs_proj)], axis=1)
    rl_tab = jnp.concatenate([_pack_i32(rela_embed.astype(jnp.bfloat16)),
                              _pack_i32(rl_proj)], axis=1)
    qp_i = _pack_i32(qp_proj)

    cq = pl.kernel(
        _sc_cq_kernel,
        out_type=jax.ShapeDtypeStruct((64, 32), jnp.int32),
        mesh=plsc.VectorSubcoreMesh(core_axis_name="c", subcore_axis_name="s"),
        scratch_types=[
            pltpu.VMEM((64,), jnp.int32),
            pltpu.VMEM((64, 32), jnp.int32),
            pltpu.SemaphoreType.DMA,
        ],
        compiler_params=pltpu.CompilerParams(use_tc_tiling_on_sc=False,
                                             needs_layout_passes=False),
    )(qrel_i, qp_i)

    sc = pl.kernel(
        functools.partial(_sc_edge_kernel, n, n_chunk),
        out_type=jax.ShapeDtypeStruct((NC * n, d), jnp.float32),
        mesh=plsc.VectorSubcoreMesh(core_axis_name="c", subcore_axis_name="s"),
        scratch_types=[
            pltpu.VMEM((2, 4 * K), jnp.int32),
            pltpu.VMEM((2, K), jnp.int32),
            pltpu.VMEM((5 * L,), jnp.float32),
            pltpu.VMEM((2, K, 96), jnp.int32),
            pltpu.VMEM((2, K, 96), jnp.int32),
            pltpu.VMEM((2, K, 32), jnp.int32),
            pltpu.VMEM((K, d), jnp.float32),
            pltpu.VMEM_SHARED((n, d), jnp.float32),
            pltpu.SemaphoreType.DMA,
            pltpu.SemaphoreType.DMA,
            pltpu.SemaphoreType.DMA,
        ],
        compiler_params=pltpu.CompilerParams(use_tc_tiling_on_sc=False,
                                             needs_layout_passes=False),
    )
    partial_out = sc(idx4, hs_tab, rl_tab, cq, wp, jnp.zeros((n, d), jnp.float32))

    p128 = np.array(_evens_odds_perm(d), np.int32)
    w_h_perm = W_h[p128, :]
    p = partial_out.reshape(NC, n, d)
    fb = 1000
    hidden_new = pl.pallas_call(
        _final_kernel,
        grid=(n // fb,),
        in_specs=[pl.BlockSpec((NC, fb, d), lambda i: (0, i, 0)),
                  pl.BlockSpec((d, d), lambda i: (0, 0))],
        out_specs=pl.BlockSpec((fb, d), lambda i: (i, 0)),
        out_shape=jax.ShapeDtypeStruct((n, d), jnp.float32),
    )(p, w_h_perm)
    return hidden_new


# speedup vs baseline: 1.0243x; 1.0121x over previous
import functools

import jax
import jax.numpy as jnp
import numpy as np
from jax import lax
from jax.experimental import pallas as pl
from jax.experimental.pallas import tpu as pltpu
from jax.experimental.pallas import tpu_sc as plsc

NC = 2
NS = 16
NW = NC * NS
K = 80
L = 16

def _mm_bf16_kernel(x_ref, w_ref, o_ref):
    o_ref[...] = jnp.dot(x_ref[...], w_ref[...],
                         preferred_element_type=jnp.float32).astype(jnp.bfloat16)


def _rela_proj_kernel(x_ref, wr_ref, wq_ref, b_ref, or_ref, oq_ref):
    x = x_ref[...]
    or_ref[...] = jnp.dot(x, wr_ref[...],
                          preferred_element_type=jnp.float32).astype(jnp.bfloat16)
    oq_ref[...] = (jnp.dot(x, wq_ref[...], preferred_element_type=jnp.float32)
                   + b_ref[...]).astype(jnp.bfloat16)


def _final_kernel(p_ref, w_ref, o_ref):
    o_ref[...] = jnp.dot(p_ref[0] + p_ref[1], w_ref[...],
                         preferred_element_type=jnp.float32)


def _sc_cq_kernel(qrel_h, qp_h, cq_h, qrel_v, cq_v, sem):
    c = lax.axis_index("c")
    s = lax.axis_index("s")

    @pl.when(jnp.logical_and(c == 0, s == 0))
    def _():
        pltpu.sync_copy(qrel_h, qrel_v)
        pltpu.async_copy(qp_h.at[qrel_v], cq_v, sem).wait()
        pltpu.sync_copy(cq_v, cq_h)


def _halves(xi):
    a = plsc.bitcast(lax.shift_left(xi, 16), jnp.float32)
    b = plsc.bitcast(lax.bitwise_and(xi, jnp.int32(-65536)), jnp.float32)
    return a, b


def _sc_edge_kernel(n_node, n_chunk, idx4_h, hid_h, hsp_h, rle_h, rlp_h, cq_h,
                    wp_h, zero_h, out_h, idx_b, obj_s, wp_v,
                    hid_b, hsp_b, rle_b, rlp_b, qp_b, msg_b,
                    acc, sem_g, sem_i, sem_s):
    c = lax.axis_index("c")
    s = lax.axis_index("s")
    wid = s * NC + c

    rows = (n_node // NS) & ~7
    tail = n_node - NS * rows
    pltpu.sync_copy(zero_h.at[pl.ds(s * rows, rows)], acc.at[pl.ds(s * rows, rows)])
    if tail:
        @pl.when(s == 0)
        def _zero_tail():
            pltpu.sync_copy(zero_h.at[pl.ds(NS * rows, tail)],
                            acc.at[pl.ds(NS * rows, tail)])
    pltpu.sync_copy(wp_h, wp_v)
    plsc.subcore_barrier()

    wa0 = wp_v[pl.ds(0, L)]
    wb0 = wp_v[pl.ds(L, L)]
    wa1 = wp_v[pl.ds(2 * L, L)]
    wb1 = wp_v[pl.ds(3 * L, L)]
    bias = wp_v[pl.ds(4 * L, L)]

    n_mine = (n_chunk - 1 - wid) // NW + 1

    def issue_idx(slot, j):
        base = (wid + j * NW) * (4 * K)
        return pltpu.async_copy(idx4_h.at[pl.ds(base, 4 * K)],
                                idx_b.at[slot], sem_i)

    def compute(slot):
        @plsc.parallel_loop(0, K, unroll=2)
        def edge_body(e):
            zero = jnp.float32(0)
            xh0 = hsp_b[slot, e, pl.ds(0, L)]
            xr0 = rlp_b[slot, e, pl.ds(0, L)]
            xq0 = qp_b[slot, e, pl.ds(0, L)]
            ah0, bh0 = _halves(xh0)
            ar0, br0 = _halves(xr0)
            aq0, bq0 = _halves(xq0)
            pa0 = ah0 + ar0 + aq0
            pb0 = bh0 + br0 + bq0
            xh1 = hsp_b[slot, e, pl.ds(L, L)]
            xr1 = rlp_b[slot, e, pl.ds(L, L)]
            xq1 = qp_b[slot, e, pl.ds(L, L)]
            ah1, bh1 = _halves(xh1)
            ar1, br1 = _halves(xr1)
            aq1, bq1 = _halves(xq1)
            pa1 = ah1 + ar1 + aq1
            pb1 = bh1 + br1 + bq1
            t0 = jnp.maximum(pa0, zero) * wa0
            t1 = jnp.maximum(pb0, zero) * wb0
            t2 = jnp.maximum(pa1, zero) * wa1
            t3 = jnp.maximum(pb1, zero) * wb1
            tot = jnp.sum((t0 + t1) + (t2 + t3))
            x = lax.broadcast_in_dim(tot, (L,), ()) + bias
            alpha = 1.0 / (1.0 + jnp.exp(-x))
            for t in range(4):
                xh = hid_b[slot, e, pl.ds(t * L, L)]
                xr = rle_b[slot, e, pl.ds(t * L, L)]
                ah, bh = _halves(xh)
                ar, br = _halves(xr)
                msg_b[e, pl.ds(2 * t * L, L)] = ah * ar * alpha
                msg_b[e, pl.ds((2 * t + 1) * L, L)] = bh * br * alpha
        for t in range(K // L):
            obj_s[slot, pl.ds(t * L, L)] = idx_b[slot, pl.ds(3 * K + t * L, L)]

    def scatter(slot):
        return pltpu.async_copy(msg_b, acc.at[obj_s.at[slot]], sem_s, add=True)

    issue_idx(0, 0).wait()

    def chunk_body(j, carry):
        p = j & 1
        q = 1 - p
        g1 = pltpu.async_copy(hid_h.at[idx_b.at[p, pl.ds(0, K)]],
                              hid_b.at[p], sem_g)
        g2 = pltpu.async_copy(hsp_h.at[idx_b.at[p, pl.ds(0, K)]],
                              hsp_b.at[p], sem_g)
        g3 = pltpu.async_copy(rle_h.at[idx_b.at[p, pl.ds(K, K)]],
                              rle_b.at[p], sem_g)
        g4 = pltpu.async_copy(rlp_h.at[idx_b.at[p, pl.ds(K, K)]],
                              rlp_b.at[p], sem_g)
        g5 = pltpu.async_copy(cq_h.at[idx_b.at[p, pl.ds(2 * K, K)]],
                              qp_b.at[p], sem_g)

        @pl.when(j > 0)
        def _steady():
            compute(q)
            sc_h = scatter(q)

            @pl.when(j + 1 < n_mine)
            def _prefetch_idx():
                issue_idx(q, j + 1).wait()

            g1.wait()
            g2.wait()
            g3.wait()
            g4.wait()
            g5.wait()
            sc_h.wait()

        @pl.when(j == 0)
        def _first():
            @pl.when(n_mine > 1)
            def _prefetch_idx0():
                issue_idx(q, 1).wait()

            g1.wait()
            g2.wait()
            g3.wait()
            g4.wait()
            g5.wait()

        return carry

    lax.fori_loop(0, n_mine, chunk_body, 0)
    last = (n_mine - 1) & 1
    compute(last)
    scatter(last).wait()

    plsc.subcore_barrier()
    pltpu.sync_copy(acc.at[pl.ds(s * rows, rows)],
                    out_h.at[pl.ds(c * n_node + s * rows, rows)])
    if tail:
        @pl.when(s == 0)
        def _out_tail():
            pltpu.sync_copy(acc.at[pl.ds(NS * rows, tail)],
                            out_h.at[pl.ds(c * n_node + NS * rows, tail)])


def _pack_i32(x_bf16):
    r, cc = x_bf16.shape
    return lax.bitcast_convert_type(x_bf16.reshape(r, cc // 2, 2), jnp.int32)


def _evens_odds_perm(width):
    perm = []
    for t in range(width // 32):
        perm += [32 * t + 2 * k for k in range(16)]
        perm += [32 * t + 2 * k + 1 for k in range(16)]
    return perm


def kernel(q_sub, q_rel, r_idx, hidden, edges, n_node, rela_embed, Ws_attn,
           Wr_attn, Wqr_W, Wqr_b, walpha_W, walpha_b, W_h):
    del q_sub
    n, d = hidden.shape
    v = rela_embed.shape[0]
    e = r_idx.shape[0]
    assert e % K == 0
    n_chunk = e // K

    e32 = edges.astype(jnp.int32)
    sub_i = e32[:, 0]
    rel_i = e32[:, 1]
    obj_i = jnp.minimum(e32[:, 2], jnp.int32(n_node) - 1)
    ridx_i = r_idx.astype(jnp.int32)
    qrel_i = q_rel.astype(jnp.int32)
    idx4 = jnp.stack([sub_i.reshape(n_chunk, K), rel_i.reshape(n_chunk, K),
                      ridx_i.reshape(n_chunk, K), obj_i.reshape(n_chunk, K)],
                     axis=1).reshape(-1)

    p64 = np.array(_evens_odds_perm(64), np.int32)
    wp = jnp.concatenate([walpha_W[p64, 0],
                          jnp.broadcast_to(walpha_b, (L,))]).astype(jnp.float32)

    hs_proj = pl.pallas_call(
        _mm_bf16_kernel,
        grid=(10,),
        in_specs=[pl.BlockSpec((n // 10, d), lambda i: (i, 0)),
                  pl.BlockSpec((d, 64), lambda i: (0, 0))],
        out_specs=pl.BlockSpec((n // 10, 64), lambda i: (i, 0)),
        out_shape=jax.ShapeDtypeStruct((n, 64), jnp.bfloat16),
    )(hidden, Ws_attn)

    rb = 1024
    rl_proj, qp_proj = pl.pallas_call(
        _rela_proj_kernel,
        grid=(pl.cdiv(v, rb),),
        in_specs=[pl.BlockSpec((rb, d), lambda i: (i, 0)),
                  pl.BlockSpec((d, 64), lambda i: (0, 0)),
                  pl.BlockSpec((d, 64), lambda i: (0, 0)),
                  pl.BlockSpec((1, 64), lambda i: (0, 0))],
        out_specs=[pl.BlockSpec((rb, 64), lambda i: (i, 0)),
                   pl.BlockSpec((rb, 64), lambda i: (i, 0))],
        out_shape=[jax.ShapeDtypeStruct((v, 64), jnp.bfloat16),
                   jax.ShapeDtypeStruct((v, 64), jnp.bfloat16)],
    )(rela_embed, Wr_attn, Wqr_W, Wqr_b.reshape(1, 64))

    hid_i = _pack_i32(hidden.astype(jnp.bfloat16))
    rle_i = _pack_i32(rela_embed.astype(jnp.bfloat16))
    hsp_i = _pack_i32(hs_proj)
    rlp_i = _pack_i32(rl_proj)
    qp_i = _pack_i32(qp_proj)

    cq = pl.kernel(
        _sc_cq_kernel,
        out_type=jax.ShapeDtypeStruct((64, 32), jnp.int32),
        mesh=plsc.VectorSubcoreMesh(core_axis_name="c", subcore_axis_name="s"),
        scratch_types=[
            pltpu.VMEM((64,), jnp.int32),
            pltpu.VMEM((64, 32), jnp.int32),
            pltpu.SemaphoreType.DMA,
        ],
        compiler_params=pltpu.CompilerParams(use_tc_tiling_on_sc=False,
                                             needs_layout_passes=False),
    )(qrel_i, qp_i)

    sc = pl.kernel(
        functools.partial(_sc_edge_kernel, n, n_chunk),
        out_type=jax.ShapeDtypeStruct((NC * n, d), jnp.float32),
        mesh=plsc.VectorSubcoreMesh(core_axis_name="c", subcore_axis_name="s"),
        scratch_types=[
            pltpu.VMEM((2, 4 * K), jnp.int32),
            pltpu.VMEM((2, K), jnp.int32),
            pltpu.VMEM((5 * L,), jnp.float32),
            pltpu.VMEM((2, K, d // 2), jnp.int32),
            pltpu.VMEM((2, K, 32), jnp.int32),
            pltpu.VMEM((2, K, d // 2), jnp.int32),
            pltpu.VMEM((2, K, 32), jnp.int32),
            pltpu.VMEM((2, K, 32), jnp.int32),
            pltpu.VMEM((K, d), jnp.float32),
            pltpu.VMEM_SHARED((n, d), jnp.float32),
            pltpu.SemaphoreType.DMA,
            pltpu.SemaphoreType.DMA,
            pltpu.SemaphoreType.DMA,
        ],
        compiler_params=pltpu.CompilerParams(use_tc_tiling_on_sc=False,
                                             needs_layout_passes=False),
    )
    partial_out = sc(idx4, hid_i, hsp_i, rle_i, rlp_i, cq, wp,
                     jnp.zeros((n, d), jnp.float32))

    p128 = np.array(_evens_odds_perm(d), np.int32)
    w_h_perm = W_h[p128, :]
    p = partial_out.reshape(NC, n, d)
    fb = 1000
    hidden_new = pl.pallas_call(
        _final_kernel,
        grid=(n // fb,),
        in_specs=[pl.BlockSpec((NC, fb, d), lambda i: (0, i, 0)),
                  pl.BlockSpec((d, d), lambda i: (0, 0))],
        out_specs=pl.BlockSpec((fb, d), lambda i: (i, 0)),
        out_shape=jax.ShapeDtypeStruct((n, d), jnp.float32),
    )(p, w_h_perm)
    return hidden_new


# speedup vs baseline: 1.1327x; 1.1059x over previous
import functools

import jax
import jax.numpy as jnp
import numpy as np
from jax import lax
from jax.experimental import pallas as pl
from jax.experimental.pallas import tpu as pltpu
from jax.experimental.pallas import tpu_sc as plsc

NC = 2
NS = 16
NW = NC * NS
K = 80
L = 16

def _mm_bf16_kernel(x_ref, w_ref, o_ref):
    o_ref[...] = jnp.dot(x_ref[...], w_ref[...],
                         preferred_element_type=jnp.float32).astype(jnp.bfloat16)


def _rela_proj_kernel(x_ref, wr_ref, wq_ref, b_ref, or_ref, oq_ref):
    x = x_ref[...]
    or_ref[...] = jnp.dot(x, wr_ref[...],
                          preferred_element_type=jnp.float32).astype(jnp.bfloat16)
    oq_ref[...] = (jnp.dot(x, wq_ref[...], preferred_element_type=jnp.float32)
                   + b_ref[...]).astype(jnp.bfloat16)


def _final_kernel(p_ref, w_ref, o_ref):
    o_ref[...] = jnp.dot(p_ref[0] + p_ref[1], w_ref[...],
                         preferred_element_type=jnp.float32)


def _sc_cq_kernel(qrel_h, qp_h, cq_h, qrel_v, cq_v, sem):
    c = lax.axis_index("c")
    s = lax.axis_index("s")

    @pl.when(jnp.logical_and(c == 0, s == 0))
    def _():
        pltpu.sync_copy(qrel_h, qrel_v)
        pltpu.async_copy(qp_h.at[qrel_v], cq_v, sem).wait()
        pltpu.sync_copy(cq_v, cq_h)


def _halves(xi):
    a = plsc.bitcast(lax.shift_left(xi, 16), jnp.float32)
    b = plsc.bitcast(lax.bitwise_and(xi, jnp.int32(-65536)), jnp.float32)
    return a, b


def _sc_edge_kernel(n_node, n_chunk, sub_h, rel_h, ridx_h, obj_h,
                    hid_h, hsp_h, rle_h, rlp_h, cq_h,
                    wp_h, zero_h, out_h, idx_b, obj_s, wp_v,
                    hid_b, hsp_b, rle_b, rlp_b, qp_b, msg_b,
                    acc, sem_g, sem_i, sem_s):
    c = lax.axis_index("c")
    s = lax.axis_index("s")
    wid = s * NC + c

    rows = (n_node // NS) & ~7
    tail = n_node - NS * rows
    pltpu.sync_copy(zero_h.at[pl.ds(s * rows, rows)], acc.at[pl.ds(s * rows, rows)])
    if tail:
        @pl.when(s == 0)
        def _zero_tail():
            pltpu.sync_copy(zero_h.at[pl.ds(NS * rows, tail)],
                            acc.at[pl.ds(NS * rows, tail)])
    pltpu.sync_copy(wp_h, wp_v)
    plsc.subcore_barrier()

    wa0 = wp_v[pl.ds(0, L)]
    wb0 = wp_v[pl.ds(L, L)]
    wa1 = wp_v[pl.ds(2 * L, L)]
    wb1 = wp_v[pl.ds(3 * L, L)]
    bias = wp_v[pl.ds(4 * L, L)]

    n_mine = (n_chunk - 1 - wid) // NW + 1

    def issue_idx(slot, j):
        base = (wid + j * NW) * K
        h1 = pltpu.async_copy(sub_h.at[pl.ds(base, K)],
                              idx_b.at[slot, pl.ds(0, K)], sem_i)
        h2 = pltpu.async_copy(rel_h.at[pl.ds(base, K)],
                              idx_b.at[slot, pl.ds(K, K)], sem_i)
        h3 = pltpu.async_copy(ridx_h.at[pl.ds(base, K)],
                              idx_b.at[slot, pl.ds(2 * K, K)], sem_i)
        h4 = pltpu.async_copy(obj_h.at[pl.ds(base, K)],
                              idx_b.at[slot, pl.ds(3 * K, K)], sem_i)
        return h1, h2, h3, h4

    def wait_idx(hs):
        for h in hs:
            h.wait()

    def compute(slot):
        @plsc.parallel_loop(0, K, unroll=2)
        def edge_body(e):
            zero = jnp.float32(0)
            xh0 = hsp_b[slot, e, pl.ds(0, L)]
            xr0 = rlp_b[slot, e, pl.ds(0, L)]
            xq0 = qp_b[slot, e, pl.ds(0, L)]
            ah0, bh0 = _halves(xh0)
            ar0, br0 = _halves(xr0)
            aq0, bq0 = _halves(xq0)
            pa0 = ah0 + ar0 + aq0
            pb0 = bh0 + br0 + bq0
            xh1 = hsp_b[slot, e, pl.ds(L, L)]
            xr1 = rlp_b[slot, e, pl.ds(L, L)]
            xq1 = qp_b[slot, e, pl.ds(L, L)]
            ah1, bh1 = _halves(xh1)
            ar1, br1 = _halves(xr1)
            aq1, bq1 = _halves(xq1)
            pa1 = ah1 + ar1 + aq1
            pb1 = bh1 + br1 + bq1
            t0 = jnp.maximum(pa0, zero) * wa0
            t1 = jnp.maximum(pb0, zero) * wb0
            t2 = jnp.maximum(pa1, zero) * wa1
            t3 = jnp.maximum(pb1, zero) * wb1
            tot = jnp.sum((t0 + t1) + (t2 + t3))
            x = lax.broadcast_in_dim(tot, (L,), ()) + bias
            alpha = 1.0 / (1.0 + jnp.exp(-x))
            for t in range(4):
                xh = hid_b[slot, e, pl.ds(t * L, L)]
                xr = rle_b[slot, e, pl.ds(t * L, L)]
                ah, bh = _halves(xh)
                ar, br = _halves(xr)
                msg_b[e, pl.ds(2 * t * L, L)] = ah * ar * alpha
                msg_b[e, pl.ds((2 * t + 1) * L, L)] = bh * br * alpha
        for t in range(K // L):
            obj_s[slot, pl.ds(t * L, L)] = idx_b[slot, pl.ds(3 * K + t * L, L)]

    def scatter(slot):
        return pltpu.async_copy(msg_b, acc.at[obj_s.at[slot]], sem_s, add=True)

    wait_idx(issue_idx(0, 0))

    def chunk_body(j, carry):
        p = j & 1
        q = 1 - p
        g1 = pltpu.async_copy(hid_h.at[idx_b.at[p, pl.ds(0, K)]],
                              hid_b.at[p], sem_g)
        g2 = pltpu.async_copy(hsp_h.at[idx_b.at[p, pl.ds(0, K)]],
                              hsp_b.at[p], sem_g)
        g3 = pltpu.async_copy(rle_h.at[idx_b.at[p, pl.ds(K, K)]],
                              rle_b.at[p], sem_g)
        g4 = pltpu.async_copy(rlp_h.at[idx_b.at[p, pl.ds(K, K)]],
                              rlp_b.at[p], sem_g)
        g5 = pltpu.async_copy(cq_h.at[idx_b.at[p, pl.ds(2 * K, K)]],
                              qp_b.at[p], sem_g)

        @pl.when(j > 0)
        def _steady():
            compute(q)
            sc_h = scatter(q)

            @pl.when(j + 1 < n_mine)
            def _prefetch_idx():
                wait_idx(issue_idx(q, j + 1))

            g1.wait()
            g2.wait()
            g3.wait()
            g4.wait()
            g5.wait()
            sc_h.wait()

        @pl.when(j == 0)
        def _first():
            @pl.when(n_mine > 1)
            def _prefetch_idx0():
                wait_idx(issue_idx(q, 1))

            g1.wait()
            g2.wait()
            g3.wait()
            g4.wait()
            g5.wait()

        return carry

    lax.fori_loop(0, n_mine, chunk_body, 0)
    last = (n_mine - 1) & 1
    compute(last)
    scatter(last).wait()

    plsc.subcore_barrier()
    pltpu.sync_copy(acc.at[pl.ds(s * rows, rows)],
                    out_h.at[pl.ds(c * n_node + s * rows, rows)])
    if tail:
        @pl.when(s == 0)
        def _out_tail():
            pltpu.sync_copy(acc.at[pl.ds(NS * rows, tail)],
                            out_h.at[pl.ds(c * n_node + NS * rows, tail)])


def _pack_i32(x_bf16):
    r, cc = x_bf16.shape
    return lax.bitcast_convert_type(x_bf16.reshape(r, cc // 2, 2), jnp.int32)


def _evens_odds_perm(width):
    perm = []
    for t in range(width // 32):
        perm += [32 * t + 2 * k for k in range(16)]
        perm += [32 * t + 2 * k + 1 for k in range(16)]
    return perm


def kernel(q_sub, q_rel, r_idx, hidden, edges, n_node, rela_embed, Ws_attn,
           Wr_attn, Wqr_W, Wqr_b, walpha_W, walpha_b, W_h):
    del q_sub
    n, d = hidden.shape
    v = rela_embed.shape[0]
    e = r_idx.shape[0]
    assert e % K == 0
    n_chunk = e // K

    e32 = edges.astype(jnp.int32)
    sub_i = e32[:, 0]
    rel_i = e32[:, 1]
    obj_i = jnp.minimum(e32[:, 2], jnp.int32(n_node) - 1)
    ridx_i = r_idx.astype(jnp.int32)
    qrel_i = q_rel.astype(jnp.int32)

    p64 = np.array(_evens_odds_perm(64), np.int32)
    wp = jnp.concatenate([walpha_W[p64, 0],
                          jnp.broadcast_to(walpha_b, (L,))]).astype(jnp.float32)

    hs_proj = pl.pallas_call(
        _mm_bf16_kernel,
        grid=(10,),
        in_specs=[pl.BlockSpec((n // 10, d), lambda i: (i, 0)),
                  pl.BlockSpec((d, 64), lambda i: (0, 0))],
        out_specs=pl.BlockSpec((n // 10, 64), lambda i: (i, 0)),
        out_shape=jax.ShapeDtypeStruct((n, 64), jnp.bfloat16),
    )(hidden, Ws_attn)

    rb = 1024
    rl_proj, qp_proj = pl.pallas_call(
        _rela_proj_kernel,
        grid=(pl.cdiv(v, rb),),
        in_specs=[pl.BlockSpec((rb, d), lambda i: (i, 0)),
                  pl.BlockSpec((d, 64), lambda i: (0, 0)),
                  pl.BlockSpec((d, 64), lambda i: (0, 0)),
                  pl.BlockSpec((1, 64), lambda i: (0, 0))],
        out_specs=[pl.BlockSpec((rb, 64), lambda i: (i, 0)),
                   pl.BlockSpec((rb, 64), lambda i: (i, 0))],
        out_shape=[jax.ShapeDtypeStruct((v, 64), jnp.bfloat16),
                   jax.ShapeDtypeStruct((v, 64), jnp.bfloat16)],
    )(rela_embed, Wr_attn, Wqr_W, Wqr_b.reshape(1, 64))

    hid_i = _pack_i32(hidden.astype(jnp.bfloat16))
    rle_i = _pack_i32(rela_embed.astype(jnp.bfloat16))
    hsp_i = _pack_i32(hs_proj)
    rlp_i = _pack_i32(rl_proj)
    qp_i = _pack_i32(qp_proj)

    cq = pl.kernel(
        _sc_cq_kernel,
        out_type=jax.ShapeDtypeStruct((64, 32), jnp.int32),
        mesh=plsc.VectorSubcoreMesh(core_axis_name="c", subcore_axis_name="s"),
        scratch_types=[
            pltpu.VMEM((64,), jnp.int32),
            pltpu.VMEM((64, 32), jnp.int32),
            pltpu.SemaphoreType.DMA,
        ],
        compiler_params=pltpu.CompilerParams(use_tc_tiling_on_sc=False,
                                             needs_layout_passes=False),
    )(qrel_i, qp_i)

    sc = pl.kernel(
        functools.partial(_sc_edge_kernel, n, n_chunk),
        out_type=jax.ShapeDtypeStruct((NC * n, d), jnp.float32),
        mesh=plsc.VectorSubcoreMesh(core_axis_name="c", subcore_axis_name="s"),
        scratch_types=[
            pltpu.VMEM((2, 4 * K), jnp.int32),
            pltpu.VMEM((2, K), jnp.int32),
            pltpu.VMEM((5 * L,), jnp.float32),
            pltpu.VMEM((2, K, d // 2), jnp.int32),
            pltpu.VMEM((2, K, 32), jnp.int32),
            pltpu.VMEM((2, K, d // 2), jnp.int32),
            pltpu.VMEM((2, K, 32), jnp.int32),
            pltpu.VMEM((2, K, 32), jnp.int32),
            pltpu.VMEM((K, d), jnp.float32),
            pltpu.VMEM_SHARED((n, d), jnp.float32),
            pltpu.SemaphoreType.DMA,
            pltpu.SemaphoreType.DMA,
            pltpu.SemaphoreType.DMA,
        ],
        compiler_params=pltpu.CompilerParams(use_tc_tiling_on_sc=False,
                                             needs_layout_passes=False),
    )
    partial_out = sc(sub_i, rel_i, ridx_i, obj_i, hid_i, hsp_i, rle_i, rlp_i,
                     cq, wp, jnp.zeros((n, d), jnp.float32))

    p128 = np.array(_evens_odds_perm(d), np.int32)
    w_h_perm = W_h[p128, :]
    p = partial_out.reshape(NC, n, d)
    fb = 1000
    hidden_new = pl.pallas_call(
        _final_kernel,
        grid=(n // fb,),
        in_specs=[pl.BlockSpec((NC, fb, d), lambda i: (0, i, 0)),
                  pl.BlockSpec((d, d), lambda i: (0, 0))],
        out_specs=pl.BlockSpec((fb, d), lambda i: (i, 0)),
        out_shape=jax.ShapeDtypeStruct((n, d), jnp.float32),
    )(p, w_h_perm)
    return hidden_new


# speedup vs baseline: 1.1521x; 1.0171x over previous
import functools

import jax
import jax.numpy as jnp
import numpy as np
from jax import lax
from jax.experimental import pallas as pl
from jax.experimental.pallas import tpu as pltpu
from jax.experimental.pallas import tpu_sc as plsc

NC = 2
NS = 16
NW = NC * NS
K = 80
L = 16

def _mm_bf16_kernel(x_ref, w_ref, o_ref):
    o_ref[...] = jnp.dot(x_ref[...], w_ref[...],
                         preferred_element_type=jnp.float32).astype(jnp.bfloat16)


def _rela_proj_kernel(x_ref, wr_ref, wq_ref, b_ref, or_ref, oq_ref):
    x = x_ref[...]
    or_ref[...] = jnp.dot(x, wr_ref[...],
                          preferred_element_type=jnp.float32).astype(jnp.bfloat16)
    oq_ref[...] = (jnp.dot(x, wq_ref[...], preferred_element_type=jnp.float32)
                   + b_ref[...]).astype(jnp.bfloat16)


def _final_kernel(p_ref, w_ref, o_ref):
    o_ref[...] = jnp.dot(p_ref[0] + p_ref[1], w_ref[...],
                         preferred_element_type=jnp.float32)


def _halves(xi):
    a = plsc.bitcast(lax.shift_left(xi, 16), jnp.float32)
    b = plsc.bitcast(lax.bitwise_and(xi, jnp.int32(-65536)), jnp.float32)
    return a, b


def _sc_edge_kernel(n_node, n_chunk, sub_h, rel_h, ridx_h, obj_h,
                    hid_h, hsp_h, rle_h, rlp_h, cq_h,
                    wp_h, zero_h, out_h, idx_b, obj_s, wp_v,
                    hid_b, hsp_b, rle_b, rlp_b, qp_b, msg_b,
                    acc, sem_g, sem_i, sem_s):
    c = lax.axis_index("c")
    s = lax.axis_index("s")
    wid = s * NC + c

    rows = (n_node // NS) & ~7
    tail = n_node - NS * rows
    pltpu.sync_copy(zero_h.at[pl.ds(s * rows, rows)], acc.at[pl.ds(s * rows, rows)])
    if tail:
        @pl.when(s == 0)
        def _zero_tail():
            pltpu.sync_copy(zero_h.at[pl.ds(NS * rows, tail)],
                            acc.at[pl.ds(NS * rows, tail)])
    pltpu.sync_copy(wp_h, wp_v)
    plsc.subcore_barrier()

    wa0 = wp_v[pl.ds(0, L)]
    wb0 = wp_v[pl.ds(L, L)]
    wa1 = wp_v[pl.ds(2 * L, L)]
    wb1 = wp_v[pl.ds(3 * L, L)]
    bias = wp_v[pl.ds(4 * L, L)]

    n_mine = (n_chunk - 1 - wid) // NW + 1

    def issue_idx(slot, j):
        base = (wid + j * NW) * K
        h1 = pltpu.async_copy(sub_h.at[pl.ds(base, K)],
                              idx_b.at[slot, pl.ds(0, K)], sem_i)
        h2 = pltpu.async_copy(rel_h.at[pl.ds(base, K)],
                              idx_b.at[slot, pl.ds(K, K)], sem_i)
        h3 = pltpu.async_copy(ridx_h.at[pl.ds(base, K)],
                              idx_b.at[slot, pl.ds(2 * K, K)], sem_i)
        h4 = pltpu.async_copy(obj_h.at[pl.ds(base, K)],
                              idx_b.at[slot, pl.ds(3 * K, K)], sem_i)
        return h1, h2, h3, h4

    def wait_idx(hs):
        for h in hs:
            h.wait()

    def compute(slot):
        @plsc.parallel_loop(0, K, unroll=2)
        def edge_body(e):
            zero = jnp.float32(0)
            xh0 = hsp_b[slot, e, pl.ds(0, L)]
            xr0 = rlp_b[slot, e, pl.ds(0, L)]
            xq0 = qp_b[slot, e, pl.ds(0, L)]
            ah0, bh0 = _halves(xh0)
            ar0, br0 = _halves(xr0)
            aq0, bq0 = _halves(xq0)
            pa0 = ah0 + ar0 + aq0
            pb0 = bh0 + br0 + bq0
            xh1 = hsp_b[slot, e, pl.ds(L, L)]
            xr1 = rlp_b[slot, e, pl.ds(L, L)]
            xq1 = qp_b[slot, e, pl.ds(L, L)]
            ah1, bh1 = _halves(xh1)
            ar1, br1 = _halves(xr1)
            aq1, bq1 = _halves(xq1)
            pa1 = ah1 + ar1 + aq1
            pb1 = bh1 + br1 + bq1
            t0 = jnp.maximum(pa0, zero) * wa0
            t1 = jnp.maximum(pb0, zero) * wb0
            t2 = jnp.maximum(pa1, zero) * wa1
            t3 = jnp.maximum(pb1, zero) * wb1
            tot = jnp.sum((t0 + t1) + (t2 + t3))
            x = lax.broadcast_in_dim(tot, (L,), ()) + bias
            alpha = 1.0 / (1.0 + jnp.exp(-x))
            for t in range(4):
                xh = hid_b[slot, e, pl.ds(t * L, L)]
                xr = rle_b[slot, e, pl.ds(t * L, L)]
                ah, bh = _halves(xh)
                ar, br = _halves(xr)
                msg_b[e, pl.ds(2 * t * L, L)] = ah * ar * alpha
                msg_b[e, pl.ds((2 * t + 1) * L, L)] = bh * br * alpha
        for t in range(K // L):
            obj_s[slot, pl.ds(t * L, L)] = idx_b[slot, pl.ds(3 * K + t * L, L)]

    def scatter(slot):
        return pltpu.async_copy(msg_b, acc.at[obj_s.at[slot]], sem_s, add=True)

    wait_idx(issue_idx(0, 0))

    def chunk_body(j, carry):
        p = j & 1
        q = 1 - p
        g1 = pltpu.async_copy(hid_h.at[idx_b.at[p, pl.ds(0, K)]],
                              hid_b.at[p], sem_g)
        g2 = pltpu.async_copy(hsp_h.at[idx_b.at[p, pl.ds(0, K)]],
                              hsp_b.at[p], sem_g)
        g3 = pltpu.async_copy(rle_h.at[idx_b.at[p, pl.ds(K, K)]],
                              rle_b.at[p], sem_g)
        g4 = pltpu.async_copy(rlp_h.at[idx_b.at[p, pl.ds(K, K)]],
                              rlp_b.at[p], sem_g)
        g5 = pltpu.async_copy(cq_h.at[idx_b.at[p, pl.ds(2 * K, K)]],
                              qp_b.at[p], sem_g)

        @pl.when(j > 0)
        def _steady():
            compute(q)
            sc_h = scatter(q)

            @pl.when(j + 1 < n_mine)
            def _prefetch_idx():
                wait_idx(issue_idx(q, j + 1))

            g1.wait()
            g2.wait()
            g3.wait()
            g4.wait()
            g5.wait()
            sc_h.wait()

        @pl.when(j == 0)
        def _first():
            @pl.when(n_mine > 1)
            def _prefetch_idx0():
                wait_idx(issue_idx(q, 1))

            g1.wait()
            g2.wait()
            g3.wait()
            g4.wait()
            g5.wait()

        return carry

    lax.fori_loop(0, n_mine, chunk_body, 0)
    last = (n_mine - 1) & 1
    compute(last)
    scatter(last).wait()

    plsc.subcore_barrier()
    pltpu.sync_copy(acc.at[pl.ds(s * rows, rows)],
                    out_h.at[pl.ds(c * n_node + s * rows, rows)])
    if tail:
        @pl.when(s == 0)
        def _out_tail():
            pltpu.sync_copy(acc.at[pl.ds(NS * rows, tail)],
                            out_h.at[pl.ds(c * n_node + NS * rows, tail)])


def _pack_i32(x_bf16):
    r, cc = x_bf16.shape
    return lax.bitcast_convert_type(x_bf16.reshape(r, cc // 2, 2), jnp.int32)


def _evens_odds_perm(width):
    perm = []
    for t in range(width // 32):
        perm += [32 * t + 2 * k for k in range(16)]
        perm += [32 * t + 2 * k + 1 for k in range(16)]
    return perm


def kernel(q_sub, q_rel, r_idx, hidden, edges, n_node, rela_embed, Ws_attn,
           Wr_attn, Wqr_W, Wqr_b, walpha_W, walpha_b, W_h):
    del q_sub
    n, d = hidden.shape
    v = rela_embed.shape[0]
    e = r_idx.shape[0]
    assert e % K == 0
    n_chunk = e // K

    e32 = edges.astype(jnp.int32)
    sub_i = e32[:, 0]
    rel_i = e32[:, 1]
    obj_i = jnp.minimum(e32[:, 2], jnp.int32(n_node) - 1)
    ridx_i = r_idx.astype(jnp.int32)
    qrel_i = q_rel.astype(jnp.int32)

    p64 = np.array(_evens_odds_perm(64), np.int32)
    wp = jnp.concatenate([walpha_W[p64, 0],
                          jnp.broadcast_to(walpha_b, (L,))]).astype(jnp.float32)

    hs_proj = pl.pallas_call(
        _mm_bf16_kernel,
        grid=(10,),
        in_specs=[pl.BlockSpec((n // 10, d), lambda i: (i, 0)),
                  pl.BlockSpec((d, 64), lambda i: (0, 0))],
        out_specs=pl.BlockSpec((n // 10, 64), lambda i: (i, 0)),
        out_shape=jax.ShapeDtypeStruct((n, 64), jnp.bfloat16),
    )(hidden, Ws_attn)

    rb = 1024
    rl_proj, qp_proj = pl.pallas_call(
        _rela_proj_kernel,
        grid=(pl.cdiv(v, rb),),
        in_specs=[pl.BlockSpec((rb, d), lambda i: (i, 0)),
                  pl.BlockSpec((d, 64), lambda i: (0, 0)),
                  pl.BlockSpec((d, 64), lambda i: (0, 0)),
                  pl.BlockSpec((1, 64), lambda i: (0, 0))],
        out_specs=[pl.BlockSpec((rb, 64), lambda i: (i, 0)),
                   pl.BlockSpec((rb, 64), lambda i: (i, 0))],
        out_shape=[jax.ShapeDtypeStruct((v, 64), jnp.bfloat16),
                   jax.ShapeDtypeStruct((v, 64), jnp.bfloat16)],
    )(rela_embed, Wr_attn, Wqr_W, Wqr_b.reshape(1, 64))

    hid_i = _pack_i32(hidden.astype(jnp.bfloat16))
    rle_i = _pack_i32(rela_embed.astype(jnp.bfloat16))
    hsp_i = _pack_i32(hs_proj)
    rlp_i = _pack_i32(rl_proj)
    qp_i = _pack_i32(qp_proj)

    cq = qp_i[qrel_i]

    sc = pl.kernel(
        functools.partial(_sc_edge_kernel, n, n_chunk),
        out_type=jax.ShapeDtypeStruct((NC * n, d), jnp.float32),
        mesh=plsc.VectorSubcoreMesh(core_axis_name="c", subcore_axis_name="s"),
        scratch_types=[
            pltpu.VMEM((2, 4 * K), jnp.int32),
            pltpu.VMEM((2, K), jnp.int32),
            pltpu.VMEM((5 * L,), jnp.float32),
            pltpu.VMEM((2, K, d // 2), jnp.int32),
            pltpu.VMEM((2, K, 32), jnp.int32),
            pltpu.VMEM((2, K, d // 2), jnp.int32),
            pltpu.VMEM((2, K, 32), jnp.int32),
            pltpu.VMEM((2, K, 32), jnp.int32),
            pltpu.VMEM((K, d), jnp.float32),
            pltpu.VMEM_SHARED((n, d), jnp.float32),
            pltpu.SemaphoreType.DMA,
            pltpu.SemaphoreType.DMA,
            pltpu.SemaphoreType.DMA,
        ],
        compiler_params=pltpu.CompilerParams(use_tc_tiling_on_sc=False,
                                             needs_layout_passes=False),
    )
    partial_out = sc(sub_i, rel_i, ridx_i, obj_i, hid_i, hsp_i, rle_i, rlp_i,
                     cq, wp, jnp.zeros((n, d), jnp.float32))

    p128 = np.array(_evens_odds_perm(d), np.int32)
    w_h_perm = W_h[p128, :]
    p = partial_out.reshape(NC, n, d)
    fb = 1000
    hidden_new = pl.pallas_call(
        _final_kernel,
        grid=(n // fb,),
        in_specs=[pl.BlockSpec((NC, fb, d), lambda i: (0, i, 0)),
                  pl.BlockSpec((d, d), lambda i: (0, 0))],
        out_specs=pl.BlockSpec((fb, d), lambda i: (i, 0)),
        out_shape=jax.ShapeDtypeStruct((n, d), jnp.float32),
    )(p, w_h_perm)
    return hidden_new


# speedup vs baseline: 1.1624x; 1.0090x over previous
import functools

import jax
import jax.numpy as jnp
import numpy as np
from jax import lax
from jax.experimental import pallas as pl
from jax.experimental.pallas import tpu as pltpu
from jax.experimental.pallas import tpu_sc as plsc

NC = 2
NS = 16
NW = NC * NS
K = 80
L = 16

def _tables_kernel(h_ref, r_ref, ws_ref, wr_ref, wq_ref, b_ref,
                   hsp_ref, rlp_ref, qpp_ref, hb_ref, rb_ref, z_ref):
    h = h_ref[...]
    r = r_ref[...]
    hsp_ref[...] = jnp.dot(h, ws_ref[...],
                           preferred_element_type=jnp.float32).astype(jnp.bfloat16)
    rlp_ref[...] = jnp.dot(r, wr_ref[...],
                           preferred_element_type=jnp.float32).astype(jnp.bfloat16)
    qpp_ref[...] = (jnp.dot(r, wq_ref[...], preferred_element_type=jnp.float32)
                    + b_ref[...]).astype(jnp.bfloat16)
    hb_ref[...] = h.astype(jnp.bfloat16)
    rb_ref[...] = r.astype(jnp.bfloat16)
    z_ref[...] = jnp.zeros_like(z_ref)


def _final_kernel(p_ref, w_ref, o_ref):
    o_ref[...] = jnp.dot(p_ref[0] + p_ref[1], w_ref[...],
                         preferred_element_type=jnp.float32)


def _halves(xi):
    a = plsc.bitcast(lax.shift_left(xi, 16), jnp.float32)
    b = plsc.bitcast(lax.bitwise_and(xi, jnp.int32(-65536)), jnp.float32)
    return a, b


def _sc_edge_kernel(n_node, n_chunk, sub_h, rel_h, ridx_h, obj_h,
                    hid_h, hsp_h, rle_h, rlp_h, cq_h,
                    wp_h, zero_h, out_h, idx_b, obj_s, wp_v,
                    hid_b, hsp_b, rle_b, rlp_b, qp_b, msg_b,
                    acc, sem_g, sem_i, sem_s):
    c = lax.axis_index("c")
    s = lax.axis_index("s")
    wid = s * NC + c

    rows = (n_node // NS) & ~7
    tail = n_node - NS * rows
    pltpu.sync_copy(zero_h.at[pl.ds(s * rows, rows)], acc.at[pl.ds(s * rows, rows)])
    if tail:
        @pl.when(s == 0)
        def _zero_tail():
            pltpu.sync_copy(zero_h.at[pl.ds(NS * rows, tail)],
                            acc.at[pl.ds(NS * rows, tail)])
    pltpu.sync_copy(wp_h, wp_v)
    plsc.subcore_barrier()

    wa0 = wp_v[pl.ds(0, L)]
    wb0 = wp_v[pl.ds(L, L)]
    wa1 = wp_v[pl.ds(2 * L, L)]
    wb1 = wp_v[pl.ds(3 * L, L)]
    bias = wp_v[pl.ds(4 * L, L)]

    n_mine = (n_chunk - 1 - wid) // NW + 1

    def issue_idx(slot, j):
        base = (wid + j * NW) * K
        h1 = pltpu.async_copy(sub_h.at[pl.ds(base, K)],
                              idx_b.at[slot, pl.ds(0, K)], sem_i)
        h2 = pltpu.async_copy(rel_h.at[pl.ds(base, K)],
                              idx_b.at[slot, pl.ds(K, K)], sem_i)
        h3 = pltpu.async_copy(ridx_h.at[pl.ds(base, K)],
                              idx_b.at[slot, pl.ds(2 * K, K)], sem_i)
        h4 = pltpu.async_copy(obj_h.at[pl.ds(base, K)],
                              idx_b.at[slot, pl.ds(3 * K, K)], sem_i)
        return h1, h2, h3, h4

    def wait_idx(hs):
        for h in hs:
            h.wait()

    def compute(slot):
        @plsc.parallel_loop(0, K, unroll=2)
        def edge_body(e):
            zero = jnp.float32(0)
            xh0 = hsp_b[slot, e, pl.ds(0, L)]
            xr0 = rlp_b[slot, e, pl.ds(0, L)]
            xq0 = qp_b[slot, e, pl.ds(0, L)]
            ah0, bh0 = _halves(xh0)
            ar0, br0 = _halves(xr0)
            aq0, bq0 = _halves(xq0)
            pa0 = ah0 + ar0 + aq0
            pb0 = bh0 + br0 + bq0
            xh1 = hsp_b[slot, e, pl.ds(L, L)]
            xr1 = rlp_b[slot, e, pl.ds(L, L)]
            xq1 = qp_b[slot, e, pl.ds(L, L)]
            ah1, bh1 = _halves(xh1)
            ar1, br1 = _halves(xr1)
            aq1, bq1 = _halves(xq1)
            pa1 = ah1 + ar1 + aq1
            pb1 = bh1 + br1 + bq1
            t0 = jnp.maximum(pa0, zero) * wa0
            t1 = jnp.maximum(pb0, zero) * wb0
            t2 = jnp.maximum(pa1, zero) * wa1
            t3 = jnp.maximum(pb1, zero) * wb1
            tot = jnp.sum((t0 + t1) + (t2 + t3))
            x = lax.broadcast_in_dim(tot, (L,), ()) + bias
            alpha = 1.0 / (1.0 + jnp.exp(-x))
            for t in range(4):
                xh = hid_b[slot, e, pl.ds(t * L, L)]
                xr = rle_b[slot, e, pl.ds(t * L, L)]
                ah, bh = _halves(xh)
                ar, br = _halves(xr)
                msg_b[e, pl.ds(2 * t * L, L)] = ah * ar * alpha
                msg_b[e, pl.ds((2 * t + 1) * L, L)] = bh * br * alpha
        for t in range(K // L):
            obj_s[slot, pl.ds(t * L, L)] = idx_b[slot, pl.ds(3 * K + t * L, L)]

    def scatter(slot):
        return pltpu.async_copy(msg_b, acc.at[obj_s.at[slot]], sem_s, add=True)

    wait_idx(issue_idx(0, 0))

    def chunk_body(j, carry):
        p = j & 1
        q = 1 - p
        g1 = pltpu.async_copy(hid_h.at[idx_b.at[p, pl.ds(0, K)]],
                              hid_b.at[p], sem_g)
        g2 = pltpu.async_copy(hsp_h.at[idx_b.at[p, pl.ds(0, K)]],
                              hsp_b.at[p], sem_g)
        g3 = pltpu.async_copy(rle_h.at[idx_b.at[p, pl.ds(K, K)]],
                              rle_b.at[p], sem_g)
        g4 = pltpu.async_copy(rlp_h.at[idx_b.at[p, pl.ds(K, K)]],
                              rlp_b.at[p], sem_g)
        g5 = pltpu.async_copy(cq_h.at[idx_b.at[p, pl.ds(2 * K, K)]],
                              qp_b.at[p], sem_g)

        @pl.when(j > 0)
        def _steady():
            compute(q)
            sc_h = scatter(q)

            @pl.when(j + 1 < n_mine)
            def _prefetch_idx():
                wait_idx(issue_idx(q, j + 1))

            g1.wait()
            g2.wait()
            g3.wait()
            g4.wait()
            g5.wait()
            sc_h.wait()

        @pl.when(j == 0)
        def _first():
            @pl.when(n_mine > 1)
            def _prefetch_idx0():
                wait_idx(issue_idx(q, 1))

            g1.wait()
            g2.wait()
            g3.wait()
            g4.wait()
            g5.wait()

        return carry

    lax.fori_loop(0, n_mine, chunk_body, 0)
    last = (n_mine - 1) & 1
    compute(last)
    scatter(last).wait()

    plsc.subcore_barrier()
    pltpu.sync_copy(acc.at[pl.ds(s * rows, rows)],
                    out_h.at[pl.ds(c * n_node + s * rows, rows)])
    if tail:
        @pl.when(s == 0)
        def _out_tail():
            pltpu.sync_copy(acc.at[pl.ds(NS * rows, tail)],
                            out_h.at[pl.ds(c * n_node + NS * rows, tail)])


def _pack_i32(x_bf16):
    r, cc = x_bf16.shape
    return lax.bitcast_convert_type(x_bf16.reshape(r, cc // 2, 2), jnp.int32)


def _evens_odds_perm(width):
    perm = []
    for t in range(width // 32):
        perm += [32 * t + 2 * k for k in range(16)]
        perm += [32 * t + 2 * k + 1 for k in range(16)]
    return perm


def kernel(q_sub, q_rel, r_idx, hidden, edges, n_node, rela_embed, Ws_attn,
           Wr_attn, Wqr_W, Wqr_b, walpha_W, walpha_b, W_h):
    del q_sub
    n, d = hidden.shape
    v = rela_embed.shape[0]
    e = r_idx.shape[0]
    assert e % K == 0
    n_chunk = e // K

    e32 = edges.astype(jnp.int32)
    sub_i = e32[:, 0]
    rel_i = e32[:, 1]
    obj_i = jnp.minimum(e32[:, 2], jnp.int32(n_node) - 1)
    ridx_i = r_idx.astype(jnp.int32)
    qrel_i = q_rel.astype(jnp.int32)

    p64 = np.array(_evens_odds_perm(64), np.int32)
    wp = jnp.concatenate([walpha_W[p64, 0],
                          jnp.broadcast_to(walpha_b, (L,))]).astype(jnp.float32)

    nb = n // 10
    rb = 1024
    hs_proj, rl_proj, qp_proj, hid_bf, rle_bf, zero = pl.pallas_call(
        _tables_kernel,
        grid=(10,),
        in_specs=[pl.BlockSpec((nb, d), lambda i: (i, 0)),
                  pl.BlockSpec((rb, d), lambda i: (i, 0)),
                  pl.BlockSpec((d, 64), lambda i: (0, 0)),
                  pl.BlockSpec((d, 64), lambda i: (0, 0)),
                  pl.BlockSpec((d, 64), lambda i: (0, 0)),
                  pl.BlockSpec((1, 64), lambda i: (0, 0))],
        out_specs=[pl.BlockSpec((nb, 64), lambda i: (i, 0)),
                   pl.BlockSpec((rb, 64), lambda i: (i, 0)),
                   pl.BlockSpec((rb, 64), lambda i: (i, 0)),
                   pl.BlockSpec((nb, d), lambda i: (i, 0)),
                   pl.BlockSpec((rb, d), lambda i: (i, 0)),
                   pl.BlockSpec((nb, d), lambda i: (i, 0))],
        out_shape=[jax.ShapeDtypeStruct((n, 64), jnp.bfloat16),
                   jax.ShapeDtypeStruct((v, 64), jnp.bfloat16),
                   jax.ShapeDtypeStruct((v, 64), jnp.bfloat16),
                   jax.ShapeDtypeStruct((n, d), jnp.bfloat16),
                   jax.ShapeDtypeStruct((v, d), jnp.bfloat16),
                   jax.ShapeDtypeStruct((n, d), jnp.float32)],
    )(hidden, rela_embed, Ws_attn, Wr_attn, Wqr_W, Wqr_b.reshape(1, 64))

    hid_i = _pack_i32(hid_bf)
    rle_i = _pack_i32(rle_bf)
    hsp_i = _pack_i32(hs_proj)
    rlp_i = _pack_i32(rl_proj)
    qp_i = _pack_i32(qp_proj)

    cq = qp_i[qrel_i]

    sc = pl.kernel(
        functools.partial(_sc_edge_kernel, n, n_chunk),
        out_type=jax.ShapeDtypeStruct((NC * n, d), jnp.float32),
        mesh=plsc.VectorSubcoreMesh(core_axis_name="c", subcore_axis_name="s"),
        scratch_types=[
            pltpu.VMEM((2, 4 * K), jnp.int32),
            pltpu.VMEM((2, K), jnp.int32),
            pltpu.VMEM((5 * L,), jnp.float32),
            pltpu.VMEM((2, K, d // 2), jnp.int32),
            pltpu.VMEM((2, K, 32), jnp.int32),
            pltpu.VMEM((2, K, d // 2), jnp.int32),
            pltpu.VMEM((2, K, 32), jnp.int32),
            pltpu.VMEM((2, K, 32), jnp.int32),
            pltpu.VMEM((K, d), jnp.float32),
            pltpu.VMEM_SHARED((n, d), jnp.float32),
            pltpu.SemaphoreType.DMA,
            pltpu.SemaphoreType.DMA,
            pltpu.SemaphoreType.DMA,
        ],
        compiler_params=pltpu.CompilerParams(use_tc_tiling_on_sc=False,
                                             needs_layout_passes=False),
    )
    partial_out = sc(sub_i, rel_i, ridx_i, obj_i, hid_i, hsp_i, rle_i, rlp_i,
                     cq, wp, zero)

    p128 = np.array(_evens_odds_perm(d), np.int32)
    w_h_perm = W_h[p128, :]
    p = partial_out.reshape(NC, n, d)
    fb = 1000
    hidden_new = pl.pallas_call(
        _final_kernel,
        grid=(n // fb,),
        in_specs=[pl.BlockSpec((NC, fb, d), lambda i: (0, i, 0)),
                  pl.BlockSpec((d, d), lambda i: (0, 0))],
        out_specs=pl.BlockSpec((fb, d), lambda i: (i, 0)),
        out_shape=jax.ShapeDtypeStruct((n, d), jnp.float32),
    )(p, w_h_perm)
    return hidden_new
